# baseline (speedup 1.0000x reference)
import functools
import jax
import jax.numpy as jnp
from jax import lax
from jax.experimental import pallas as pl
from jax.experimental.pallas import tpu as pltpu
from jax.experimental.pallas import tpu_sc as plsc

_NC, _NS = 2, 16
_NW = _NC * _NS
_CK = 512


def kernel(x, table):
    B, L, D = x.shape
    E = table.shape[-1]
    W = D + E

    xt = jnp.transpose(x, (1, 2, 0))
    ep = jnp.broadcast_to(table[:, :, None], (L, E, 128))

    mesh = plsc.VectorSubcoreMesh(core_axis_name="c", subcore_axis_name="s",
                                  num_cores=_NC)

    @functools.partial(
        pl.kernel, mesh=mesh,
        out_type=jax.ShapeDtypeStruct((L, W, B), jnp.float32),
        scratch_types=[
            pltpu.VMEM((D, _CK), jnp.float32),
            pltpu.VMEM((E, 128), jnp.float32),
            pltpu.SemaphoreType.DMA,
        ],
    )
    def sc_concat(x_hbm, ep_hbm, o_hbm, xv, epv, sem):
        wid = lax.axis_index("s") * _NC + lax.axis_index("c")
        for i in range((L + _NW - 1) // _NW):
            l = wid + _NW * i

            @pl.when(l < L)
            def _():
                for kk in range(B // _CK):
                    pltpu.async_copy(
                        x_hbm.at[l, :, pl.ds(kk * _CK, _CK)], xv, sem).wait()
                    pltpu.async_copy(
                        xv, o_hbm.at[l, pl.ds(0, D), pl.ds(kk * _CK, _CK)],
                        sem).wait()
                pltpu.async_copy(ep_hbm.at[l], epv, sem).wait()
                emb_copies = [
                    pltpu.make_async_copy(
                        epv, o_hbm.at[l, pl.ds(D, E), pl.ds(kk * 128, 128)],
                        sem)
                    for kk in range(B // 128)]
                for c in emb_copies:
                    c.start()
                for c in emb_copies:
                    c.wait()

    out_t = sc_concat(xt, ep)
    return jnp.transpose(out_t, (2, 0, 1))


# speedup vs baseline: 1.6430x; 1.6430x over previous
import jax
import jax.numpy as jnp
from jax.experimental import pallas as pl

_L_BLK = 10


def _concat_body(x_ref, ep_ref, o_ref):
    o_ref[:, :64, :] = x_ref[...]
    ep = ep_ref[...]
    o_ref[:, 64:, :] = jnp.tile(ep, (1, 1, 32))


def kernel(x, table):
    B, L, D = x.shape
    E = table.shape[-1]
    W = D + E

    xt = jnp.transpose(x, (1, 2, 0))
    ep = jnp.broadcast_to(table[:, :, None], (L, E, 128))

    out_t = pl.pallas_call(
        _concat_body,
        grid=(L // _L_BLK,),
        in_specs=[
            pl.BlockSpec((_L_BLK, D, B), lambda i: (i, 0, 0)),
            pl.BlockSpec((_L_BLK, E, 128), lambda i: (i, 0, 0)),
        ],
        out_specs=pl.BlockSpec((_L_BLK, W, B), lambda i: (i, 0, 0)),
        out_shape=jax.ShapeDtypeStruct((L, W, B), x.dtype),
    )(xt, ep)
    return jnp.transpose(out_t, (2, 0, 1))
